# baseline (speedup 1.0000x reference)
import functools

import jax
import jax.numpy as jnp
from jax import lax
from jax.experimental import pallas as pl
from jax.experimental.pallas import tpu as pltpu
from jax.experimental.pallas import tpu_sc as plsc

D = 64
NC = 2
NS = 16
NW = NC * NS
C = 128
NB = 5


@functools.lru_cache(maxsize=None)
def _build(bsz, hist, nv):
    assert bsz == NW * C and hist % NB == 0

    mesh = plsc.VectorSubcoreMesh(core_axis_name="c", subcore_axis_name="s")

    @functools.partial(
        pl.kernel,
        mesh=mesh,
        out_type=jax.ShapeDtypeStruct((hist, D, bsz), jnp.float32),
        scratch_types=(
            [pltpu.VMEM((hist, C), jnp.int32)]
            + [pltpu.VMEM((C, D), jnp.float32) for _ in range(NB)]
            + [pltpu.VMEM((D, C), jnp.float32)]
            + [pltpu.SemaphoreType.DMA for _ in range(NB)]
        ),
        compiler_params=pltpu.CompilerParams(use_tc_tiling_on_sc=False, needs_layout_passes=False),
    )
    def k(table_hbm, idxt_hbm, out_hbm, idx_v, *rest):
        bufs = rest[:NB]
        buf_t = rest[NB]
        sems = rest[NB + 1:]
        wid = lax.axis_index("s") * NC + lax.axis_index("c")
        base = wid * C

        pltpu.sync_copy(idxt_hbm.at[:, pl.ds(base, C)], idx_v)

        for b in range(NB):
            pltpu.async_copy(table_hbm.at[idx_v.at[b]], bufs[b], sems[b])

        lanes = lax.iota(jnp.int32, 16)

        def chunk(h, buf):
            def pos_group(p, carry):
                rows = p * 16 + lanes
                for f in range(D):
                    vals = plsc.load_gather(buf, [rows, jnp.full((16,), f, jnp.int32)])
                    buf_t[f, pl.ds(p * 16, 16)] = vals
                return carry

            lax.fori_loop(0, C // 16, pos_group, 0)
            pltpu.sync_copy(buf_t, out_hbm.at[h, :, pl.ds(base, C)])

        def outer(o, carry):
            for b in range(NB):
                h = o * NB + b
                pltpu.make_async_copy(
                    table_hbm.at[idx_v.at[h]], bufs[b], sems[b]
                ).wait()
                chunk(h, bufs[b])
                pltpu.async_copy(
                    table_hbm.at[idx_v.at[h + NB]], bufs[b], sems[b]
                )
            return carry

        lax.fori_loop(0, (hist - NB) // NB, outer, 0)

        for b in range(NB):
            h = hist - NB + b
            pltpu.make_async_copy(
                table_hbm.at[idx_v.at[h]], bufs[b], sems[b]
            ).wait()
            chunk(h, bufs[b])

    return k


def kernel(inputs, embedding):
    bsz, hist = inputs.shape
    nv, d = embedding.shape
    idx_t = inputs.T.astype(jnp.int32)
    out = _build(bsz, hist, nv)(embedding, idx_t)
    return out.transpose(2, 0, 1)
